# baseline (speedup 1.0000x reference)
import functools

import jax
import jax.numpy as jnp
from jax import lax
from jax.experimental import pallas as pl

S0, P0, S1, D = 2048, 64, 1024, 64
SB = 32
NSLOT = 3 * SB
G = S0 // SB
NS = 3 * S0
CH = 512
NCH = NS // CH
NEG = -1e30


def _rowT(v):
    n = v.shape[1]
    eye = (lax.broadcasted_iota(jnp.int32, (n, n), 0)
           == lax.broadcasted_iota(jnp.int32, (n, n), 1)).astype(jnp.float32)
    return lax.dot_general(eye, v, (((1,), (1,)), ((), ())),
                           preferred_element_type=jnp.float32, precision=lax.Precision.HIGHEST)


def _pass1_body(pf_ref, rf_ref, hc_ref, spf_ref, spc_ref, spi_ref,
                w_ref, asg_ref, pfeat_ref, aux_ref):
    pf = pf_ref[...]
    rf = rf_ref[...]
    hc = hc_ref[...]
    spf = spf_ref[...]
    spc = spc_ref[...]
    spi = spi_ref[0]
    w = w_ref[...]

    npts = SB * P0

    iota_v = lax.broadcasted_iota(jnp.int32, (S1, NSLOT), 0)
    onehotT = (iota_v == spi).astype(jnp.float32)
    cf = lax.dot_general(onehotT, spf, (((0,), (0,)), ((), ())),
                         preferred_element_type=jnp.float32,
                         precision=lax.Precision.HIGHEST)
    cc = lax.dot_general(onehotT, spc, (((0,), (0,)), ((), ())),
                         preferred_element_type=jnp.float32,
                         precision=lax.Precision.HIGHEST)
    cwT = lax.dot_general(w, cc, (((1,), (1,)), ((), ())),
                          preferred_element_type=jnp.float32,
                          precision=lax.Precision.HIGHEST)

    sims = lax.dot_general(pf, cf, (((1,), (1,)), ((), ())),
                           preferred_element_type=jnp.float32,
                           precision=lax.Precision.HIGHEST)
    rawdots = lax.dot_general(rf, cf, (((1,), (1,)), ((), ())),
                              preferred_element_type=jnp.float32,
                              precision=lax.Precision.HIGHEST)

    row_i = lax.broadcasted_iota(jnp.int32, (npts, NSLOT), 0)
    col_j = lax.broadcasted_iota(jnp.int32, (npts, NSLOT), 1)
    valid = (col_j // 3) == (row_i // P0)

    s_m = jnp.where(valid, sims, -jnp.float32(3e38))
    ibits = lax.bitcast_convert_type(s_m, jnp.int32)
    mono = jnp.where(ibits < 0, ibits ^ jnp.int32(0x7FFFFFFF), ibits)
    key = (mono & jnp.int32(~3)) + (2 - col_j % 3)
    mkey = jnp.max(key, axis=1, keepdims=True)
    msel = key == mkey
    mself = msel.astype(jnp.float32)

    spi_f = spi.astype(jnp.float32)
    spi_col = _rowT(spi_f)
    asg_f = lax.dot_general(mself, spi_col, (((1,), (0,)), ((), ())),
                            preferred_element_type=jnp.float32,
                            precision=lax.Precision.HIGHEST)
    assigned = asg_f.astype(jnp.int32)
    asg_ref[...] = assigned.reshape(SB, P0)

    ones_col = jnp.ones((NSLOT, 1), jnp.float32)
    hw = lax.dot_general(hc, w, (((1,), (1,)), ((), ())),
                         preferred_element_type=jnp.float32,
                         precision=lax.Precision.HIGHEST)
    combo = rawdots - cwT
    combo_b = lax.dot_general(mself * combo, ones_col, (((1,), (0,)), ((), ())),
                              preferred_element_type=jnp.float32,
                              precision=lax.Precision.HIGHEST)
    logit = (combo_b + hw) * jnp.float32(0.125)

    a = jnp.where(msel, logit, NEG)
    pmax = jnp.max(a, axis=0, keepdims=True)
    e = jnp.exp(jnp.where(msel, logit - pmax, NEG))

    rf_aug = jnp.concatenate([rf, jnp.ones((npts, 1), jnp.float32)], axis=1)
    pfeat_aug = lax.dot_general(e, rf_aug, (((0,), (0,)), ((), ())),
                                preferred_element_type=jnp.float32,
                                precision=lax.Precision.HIGHEST)
    pcoord = lax.dot_general(mself, hc, (((0,), (0,)), ((), ())),
                             preferred_element_type=jnp.float32,
                             precision=lax.Precision.HIGHEST)

    pfeat_ref[...] = pfeat_aug[:, :D]
    aux_ref[...] = jnp.concatenate(
        [_rowT(pmax), pfeat_aug[:, D:D + 1], pcoord[:, 3:4],
         _rowT(spi_f), pcoord[:, 0:3], jnp.zeros((NSLOT, 1), jnp.float32)],
        axis=1)


def _pass2_body(pfeat_ref, aux_ref, g_ref, b_ref, feat_ref, coord_ref):
    iota_j = lax.broadcasted_iota(jnp.int32, (CH, S1), 1)

    def phase_a(c, m):
        aux = aux_ref[pl.ds(c * CH, CH), :]
        pmax = aux[:, 0:1]
        tgt = aux[:, 3:4].astype(jnp.int32)
        o = iota_j == tgt
        cand = jnp.where(o, pmax, NEG)
        return jnp.maximum(m, jnp.max(cand, axis=0, keepdims=True))

    m = lax.fori_loop(0, NCH, phase_a, jnp.full((1, S1), NEG, jnp.float32))
    m0 = jnp.where(m > jnp.float32(-1e29), m, 0.0)

    def phase_b(c, acc):
        aux = aux_ref[pl.ds(c * CH, CH), :]
        pfeat = pfeat_ref[pl.ds(c * CH, CH), :]
        pmax = aux[:, 0:1]
        psum = aux[:, 1:2]
        pcount = aux[:, 2:3]
        tgt = aux[:, 3:4].astype(jnp.int32)
        pcoord = aux[:, 4:8]
        o = iota_j == tgt
        of = o.astype(jnp.float32)
        m0g = jnp.max(jnp.where(o, m0, NEG), axis=1, keepdims=True)
        scale = jnp.exp(jnp.where(pmax > jnp.float32(-1e29), pmax - m0g, NEG))
        faug = jnp.concatenate(
            [pfeat * scale, psum * scale, pcount, pcoord], axis=1)
        return acc + lax.dot_general(of, faug, (((0,), (0,)), ((), ())),
                                     preferred_element_type=jnp.float32, precision=lax.Precision.HIGHEST)

    acc = lax.fori_loop(0, NCH, phase_b,
                        jnp.zeros((S1, D + 6), jnp.float32))

    featsum = acc[:, :D]
    den = acc[:, D:D + 1]
    cnt = acc[:, D + 1:D + 2]
    csum = acc[:, D + 2:D + 6]

    sp_feat = featsum / (den + 1e-9)
    mu = jnp.mean(sp_feat, axis=1, keepdims=True)
    xc = sp_feat - mu
    var = jnp.mean(xc * xc, axis=1, keepdims=True)
    feat_ref[...] = xc / jnp.sqrt(var + 1e-5) * g_ref[...] + b_ref[...]
    coord_ref[...] = csum / jnp.maximum(cnt, 1.0)


@jax.jit
def _run(pf, rf, hc4, spf, spc4, spi3d, w4, gamma, beta):
    n = S0 * P0
    asg, pfeat, aux = pl.pallas_call(
        _pass1_body,
        grid=(G,),
        in_specs=[
            pl.BlockSpec((SB * P0, D), lambda i: (i, 0)),
            pl.BlockSpec((SB * P0, D), lambda i: (i, 0)),
            pl.BlockSpec((SB * P0, 4), lambda i: (i, 0)),
            pl.BlockSpec((S1, D), lambda i: (0, 0)),
            pl.BlockSpec((S1, 4), lambda i: (0, 0)),
            pl.BlockSpec((1, 1, NSLOT), lambda i: (i, 0, 0)),
            pl.BlockSpec((1, 4), lambda i: (0, 0)),
        ],
        out_specs=[
            pl.BlockSpec((SB, P0), lambda i: (i, 0)),
            pl.BlockSpec((NSLOT, D), lambda i: (i, 0)),
            pl.BlockSpec((NSLOT, 8), lambda i: (i, 0)),
        ],
        out_shape=[
            jax.ShapeDtypeStruct((S0, P0), jnp.int32),
            jax.ShapeDtypeStruct((NS, D), jnp.float32),
            jax.ShapeDtypeStruct((NS, 8), jnp.float32),
        ],
    )(pf, rf, hc4, spf, spc4, spi3d, w4)

    sp_feat, sp_coord4 = pl.pallas_call(
        _pass2_body,
        out_shape=[
            jax.ShapeDtypeStruct((S1, D), jnp.float32),
            jax.ShapeDtypeStruct((S1, 4), jnp.float32),
        ],
    )(pfeat, aux, gamma, beta)

    return asg.reshape(n), sp_feat, sp_coord4[:, :3]


def kernel(rawPoint_feat, rawPoint_coord, hilbert_feat_coord, points_feat,
           points_coord, sp_center_feat, sp_center_coord, w_rpe, ln_gamma,
           ln_beta, level0_to_level1_indices, num_segments0,
           points_per_segment0, num_segments1, points_per_segment1,
           segments_per_level0):
    n = S0 * P0
    l2l = level0_to_level1_indices.astype(jnp.int32)
    left = jnp.concatenate([l2l[:1], l2l[:-1]])
    right = jnp.concatenate([l2l[1:], l2l[-1:]])
    spi = jnp.stack([left, l2l, right], axis=1)
    spi3d = spi.reshape(G, 1, NSLOT)

    pf = points_feat.reshape(n, D)
    hc4 = jnp.pad(hilbert_feat_coord, ((0, 0), (0, 1)), constant_values=1.0)
    spc4 = jnp.pad(sp_center_coord, ((0, 0), (0, 1)))
    w4 = jnp.pad(w_rpe, (0, 1)).reshape(1, 4)
    gamma = ln_gamma.reshape(1, D)
    beta = ln_beta.reshape(1, D)

    asg, sp_feat, sp_coord = _run(pf, rawPoint_feat, hc4,
                                  sp_center_feat, spc4, spi3d, w4,
                                  gamma, beta)

    points_feat_out = rawPoint_feat.reshape(S0, P0, D)
    hilbert_feat_level1 = rawPoint_feat.reshape(S1, P0 * 2, D)
    return (asg, sp_feat, sp_coord, points_feat_out, hilbert_feat_level1)


# speedup vs baseline: 1.1242x; 1.1242x over previous
import functools

import jax
import jax.numpy as jnp
from jax import lax
from jax.experimental import pallas as pl

S0, P0, S1, D = 2048, 64, 1024, 64
SB = 32
NSLOT = 3 * SB
G = S0 // SB
NS = 3 * S0
CH = 512
NCH = NS // CH
NEG = -1e30


def _rowT(v):
    n = v.shape[1]
    eye = (lax.broadcasted_iota(jnp.int32, (n, n), 0)
           == lax.broadcasted_iota(jnp.int32, (n, n), 1)).astype(jnp.float32)
    return lax.dot_general(eye, v, (((1,), (1,)), ((), ())),
                           preferred_element_type=jnp.float32, precision=lax.Precision.HIGHEST)


def _pass1_body(pf_ref, rf_ref, hc_ref, spf_ref, spc_ref, spi_ref,
                w_ref, asg_ref, pfeat_ref, aux_ref, cp1_ref, cp2_ref):
    pf = pf_ref[...]
    rf = rf_ref[...]
    hc = hc_ref[...]
    spf = spf_ref[...]
    spc = spc_ref[...]
    spi = spi_ref[0]
    w = w_ref[...]

    npts = SB * P0

    iota_v = lax.broadcasted_iota(jnp.int32, (S1, NSLOT), 0)
    onehotT = (iota_v == spi).astype(jnp.float32)
    cf = lax.dot_general(onehotT, spf, (((0,), (0,)), ((), ())),
                         preferred_element_type=jnp.float32,
                         precision=lax.Precision.HIGHEST)
    cc = lax.dot_general(onehotT, spc, (((0,), (0,)), ((), ())),
                         preferred_element_type=jnp.float32,
                         precision=lax.Precision.HIGHEST)
    cwT = lax.dot_general(w, cc, (((1,), (1,)), ((), ())),
                          preferred_element_type=jnp.float32,
                          precision=lax.Precision.HIGHEST)

    sims = lax.dot_general(pf, cf, (((1,), (1,)), ((), ())),
                           preferred_element_type=jnp.float32,
                           precision=lax.Precision.HIGHEST)
    rawdots = lax.dot_general(rf, cf, (((1,), (1,)), ((), ())),
                              preferred_element_type=jnp.float32,
                              precision=lax.Precision.HIGHEST)

    row_i = lax.broadcasted_iota(jnp.int32, (npts, NSLOT), 0)
    col_j = lax.broadcasted_iota(jnp.int32, (npts, NSLOT), 1)
    valid = (col_j // 3) == (row_i // P0)

    s_m = jnp.where(valid, sims, -jnp.float32(3e38))
    ibits = lax.bitcast_convert_type(s_m, jnp.int32)
    mono = jnp.where(ibits < 0, ibits ^ jnp.int32(0x7FFFFFFF), ibits)
    key = (mono & jnp.int32(~3)) + (2 - col_j % 3)
    mkey = jnp.max(key, axis=1, keepdims=True)
    msel = key == mkey
    mself = msel.astype(jnp.float32)

    spi_f = spi.astype(jnp.float32)
    spi_col = _rowT(spi_f)
    asg_f = lax.dot_general(mself, spi_col, (((1,), (0,)), ((), ())),
                            preferred_element_type=jnp.float32,
                            precision=lax.Precision.HIGHEST)
    assigned = asg_f.astype(jnp.int32)
    asg_ref[...] = assigned.reshape(SB, P0)

    ones_col = jnp.ones((NSLOT, 1), jnp.float32)
    hw = lax.dot_general(hc, w, (((1,), (1,)), ((), ())),
                         preferred_element_type=jnp.float32,
                         precision=lax.Precision.HIGHEST)
    combo = rawdots - cwT
    combo_b = lax.dot_general(mself * combo, ones_col, (((1,), (0,)), ((), ())),
                              preferred_element_type=jnp.float32,
                              precision=lax.Precision.HIGHEST)
    logit = (combo_b + hw) * jnp.float32(0.125)

    a = jnp.where(msel, logit, NEG)
    pmax = jnp.max(a, axis=0, keepdims=True)
    e = jnp.exp(jnp.where(msel, logit - pmax, NEG))

    rf_aug = jnp.concatenate([rf, jnp.ones((npts, 1), jnp.float32)], axis=1)
    pfeat_aug = lax.dot_general(e, rf_aug, (((0,), (0,)), ((), ())),
                                preferred_element_type=jnp.float32,
                                precision=lax.Precision.HIGHEST)
    pcoord = lax.dot_general(mself, hc, (((0,), (0,)), ((), ())),
                             preferred_element_type=jnp.float32,
                             precision=lax.Precision.HIGHEST)

    cp1_ref[...] = rf
    cp2_ref[...] = rf
    pfeat_ref[...] = pfeat_aug[:, :D]
    aux_ref[...] = jnp.concatenate(
        [_rowT(pmax), pfeat_aug[:, D:D + 1], pcoord[:, 3:4],
         _rowT(spi_f), pcoord[:, 0:3], jnp.zeros((NSLOT, 1), jnp.float32)],
        axis=1)


def _pass2_body(pfeat_ref, aux_ref, g_ref, b_ref, feat_ref, coord_ref):
    iota_j = lax.broadcasted_iota(jnp.int32, (CH, S1), 1)

    def phase_a(c, m):
        aux = aux_ref[pl.ds(c * CH, CH), :]
        pmax = aux[:, 0:1]
        tgt = aux[:, 3:4].astype(jnp.int32)
        o = iota_j == tgt
        cand = jnp.where(o, pmax, NEG)
        return jnp.maximum(m, jnp.max(cand, axis=0, keepdims=True))

    m = lax.fori_loop(0, NCH, phase_a, jnp.full((1, S1), NEG, jnp.float32))
    m0 = jnp.where(m > jnp.float32(-1e29), m, 0.0)

    def phase_b(c, acc):
        aux = aux_ref[pl.ds(c * CH, CH), :]
        pfeat = pfeat_ref[pl.ds(c * CH, CH), :]
        pmax = aux[:, 0:1]
        psum = aux[:, 1:2]
        pcount = aux[:, 2:3]
        tgt = aux[:, 3:4].astype(jnp.int32)
        pcoord = aux[:, 4:8]
        o = iota_j == tgt
        of = o.astype(jnp.float32)
        m0g = jnp.max(jnp.where(o, m0, NEG), axis=1, keepdims=True)
        scale = jnp.exp(jnp.where(pmax > jnp.float32(-1e29), pmax - m0g, NEG))
        faug = jnp.concatenate(
            [pfeat * scale, psum * scale, pcount, pcoord], axis=1)
        return acc + lax.dot_general(of, faug, (((0,), (0,)), ((), ())),
                                     preferred_element_type=jnp.float32, precision=lax.Precision.HIGHEST)

    acc = lax.fori_loop(0, NCH, phase_b,
                        jnp.zeros((S1, D + 6), jnp.float32))

    featsum = acc[:, :D]
    den = acc[:, D:D + 1]
    cnt = acc[:, D + 1:D + 2]
    csum = acc[:, D + 2:D + 6]

    sp_feat = featsum / (den + 1e-9)
    mu = jnp.mean(sp_feat, axis=1, keepdims=True)
    xc = sp_feat - mu
    var = jnp.mean(xc * xc, axis=1, keepdims=True)
    feat_ref[...] = xc / jnp.sqrt(var + 1e-5) * g_ref[...] + b_ref[...]
    coord_ref[...] = csum / jnp.maximum(cnt, 1.0)


@jax.jit
def _run(pf, rf, hc4, spf, spc4, spi3d, w4, gamma, beta):
    n = S0 * P0
    asg, pfeat, aux, cp1, cp2 = pl.pallas_call(
        _pass1_body,
        grid=(G,),
        in_specs=[
            pl.BlockSpec((SB * P0, D), lambda i: (i, 0)),
            pl.BlockSpec((SB * P0, D), lambda i: (i, 0)),
            pl.BlockSpec((SB * P0, 4), lambda i: (i, 0)),
            pl.BlockSpec((S1, D), lambda i: (0, 0)),
            pl.BlockSpec((S1, 4), lambda i: (0, 0)),
            pl.BlockSpec((1, 1, NSLOT), lambda i: (i, 0, 0)),
            pl.BlockSpec((1, 4), lambda i: (0, 0)),
        ],
        out_specs=[
            pl.BlockSpec((SB, P0), lambda i: (i, 0)),
            pl.BlockSpec((NSLOT, D), lambda i: (i, 0)),
            pl.BlockSpec((NSLOT, 8), lambda i: (i, 0)),
            pl.BlockSpec((SB * P0, D), lambda i: (i, 0)),
            pl.BlockSpec((SB * P0, D), lambda i: (i, 0)),
        ],
        out_shape=[
            jax.ShapeDtypeStruct((S0, P0), jnp.int32),
            jax.ShapeDtypeStruct((NS, D), jnp.float32),
            jax.ShapeDtypeStruct((NS, 8), jnp.float32),
            jax.ShapeDtypeStruct((S0 * P0, D), jnp.float32),
            jax.ShapeDtypeStruct((S0 * P0, D), jnp.float32),
        ],
    )(pf, rf, hc4, spf, spc4, spi3d, w4)

    sp_feat, sp_coord4 = pl.pallas_call(
        _pass2_body,
        out_shape=[
            jax.ShapeDtypeStruct((S1, D), jnp.float32),
            jax.ShapeDtypeStruct((S1, 4), jnp.float32),
        ],
    )(pfeat, aux, gamma, beta)

    return asg.reshape(n), sp_feat, sp_coord4[:, :3], cp1, cp2


def kernel(rawPoint_feat, rawPoint_coord, hilbert_feat_coord, points_feat,
           points_coord, sp_center_feat, sp_center_coord, w_rpe, ln_gamma,
           ln_beta, level0_to_level1_indices, num_segments0,
           points_per_segment0, num_segments1, points_per_segment1,
           segments_per_level0):
    n = S0 * P0
    l2l = level0_to_level1_indices.astype(jnp.int32)
    left = jnp.concatenate([l2l[:1], l2l[:-1]])
    right = jnp.concatenate([l2l[1:], l2l[-1:]])
    spi = jnp.stack([left, l2l, right], axis=1)
    spi3d = spi.reshape(G, 1, NSLOT)

    pf = points_feat.reshape(n, D)
    hc4 = jnp.pad(hilbert_feat_coord, ((0, 0), (0, 1)), constant_values=1.0)
    spc4 = jnp.pad(sp_center_coord, ((0, 0), (0, 1)))
    w4 = jnp.pad(w_rpe, (0, 1)).reshape(1, 4)
    gamma = ln_gamma.reshape(1, D)
    beta = ln_beta.reshape(1, D)

    asg, sp_feat, sp_coord, cp1, cp2 = _run(pf, rawPoint_feat, hc4,
                                            sp_center_feat, spc4, spi3d, w4,
                                            gamma, beta)

    points_feat_out = cp1.reshape(S0, P0, D)
    hilbert_feat_level1 = cp2.reshape(S1, P0 * 2, D)
    return (asg, sp_feat, sp_coord, points_feat_out, hilbert_feat_level1)
